# Initial kernel scaffold; baseline (speedup 1.0000x reference)
import jax
import jax.numpy as jnp
from jax.experimental import pallas as pl


def kernel(x, weight):
    raise NotImplementedError("write your pallas kernel here")


# speedup vs baseline: 1.7986x; 1.7986x over previous
import functools

import jax
import jax.numpy as jnp
from jax import lax
from jax.experimental import pallas as pl
from jax.experimental.pallas import tpu as pltpu
from jax.experimental.pallas import tpu_sc as plsc


def _build(n_flat: int, dim: int, num_workers: int, chunk: int):
    per_w = n_flat // num_workers
    n_chunks = per_w // chunk
    mesh = plsc.VectorSubcoreMesh(core_axis_name="c", subcore_axis_name="s")

    @functools.partial(
        pl.kernel,
        mesh=mesh,
        out_type=jax.ShapeDtypeStruct((n_flat, dim), jnp.float32),
        scratch_types=[
            pltpu.VMEM((chunk,), jnp.int32),
            pltpu.VMEM((chunk, dim), jnp.float32),
            pltpu.SemaphoreType.DMA,
        ],
        compiler_params=pltpu.CompilerParams(use_tc_tiling_on_sc=False),
    )
    def emb(w_hbm, x_hbm, out_hbm, idx_v, rows_v, sem):
        nc = lax.axis_size("c")
        wid = lax.axis_index("s") * nc + lax.axis_index("c")
        w_base = wid * per_w

        def body(i, carry):
            base = w_base + i * chunk
            pltpu.sync_copy(x_hbm.at[pl.ds(base, chunk)], idx_v)
            pltpu.async_copy(w_hbm.at[idx_v], rows_v, sem).wait()
            pltpu.sync_copy(rows_v, out_hbm.at[pl.ds(base, chunk)])
            return carry

        lax.fori_loop(0, n_chunks, body, 0)

    return emb


def kernel(x, weight):
    b, h = x.shape
    n_vocab, dim = weight.shape
    n_flat = b * h
    info = plsc.get_sparse_core_info()
    num_workers = info.num_cores * info.num_subcores
    chunk = 512
    emb = _build(n_flat, dim, num_workers, chunk)
    out = emb(weight, x.reshape(n_flat))
    return out.reshape(b, h, dim)


# speedup vs baseline: 1.8712x; 1.0404x over previous
import functools

import jax
import jax.numpy as jnp
from jax import lax
from jax.experimental import pallas as pl
from jax.experimental.pallas import tpu as pltpu
from jax.experimental.pallas import tpu_sc as plsc


def _build(n_flat: int, dim: int, num_workers: int, chunk: int, nbuf: int):
    per_w = n_flat // num_workers
    n_chunks = per_w // chunk
    assert per_w % chunk == 0 and n_chunks % nbuf == 0 and n_chunks > nbuf
    mesh = plsc.VectorSubcoreMesh(core_axis_name="c", subcore_axis_name="s")

    scratch = (
        [pltpu.VMEM((per_w,), jnp.int32)]
        + [pltpu.VMEM((chunk, dim), jnp.float32) for _ in range(nbuf)]
        + [pltpu.SemaphoreType.DMA for _ in range(2 * nbuf)]
    )

    @functools.partial(
        pl.kernel,
        mesh=mesh,
        out_type=jax.ShapeDtypeStruct((n_flat, dim), jnp.float32),
        scratch_types=scratch,
        compiler_params=pltpu.CompilerParams(use_tc_tiling_on_sc=False),
    )
    def emb(w_hbm, x_hbm, out_hbm, idx_v, *bufs):
        rows = bufs[:nbuf]
        gsem = bufs[nbuf : 2 * nbuf]
        ssem = bufs[2 * nbuf : 3 * nbuf]
        nc = lax.axis_size("c")
        wid = lax.axis_index("s") * nc + lax.axis_index("c")
        w_base = wid * per_w

        pltpu.sync_copy(x_hbm.at[pl.ds(w_base, per_w)], idx_v)

        def gather(i, b):
            pltpu.async_copy(
                w_hbm.at[idx_v.at[pl.ds(i * chunk, chunk)]], rows[b], gsem[b]
            )

        def gather_wait(b):
            pltpu.make_async_copy(
                out_hbm.at[pl.ds(w_base, chunk)], rows[b], gsem[b]
            ).wait()

        def store(j, b):
            pltpu.async_copy(
                rows[b], out_hbm.at[pl.ds(w_base + j * chunk, chunk)], ssem[b]
            )

        def store_wait(b):
            pltpu.make_async_copy(
                rows[b], out_hbm.at[pl.ds(w_base, chunk)], ssem[b]
            ).wait()

        for b in range(nbuf - 1):
            gather(b, b)
        gather_wait(0)
        store(0, 0)
        gather(nbuf - 1, nbuf - 1)

        def group(g, carry):
            j0 = 1 + g * nbuf
            for t in range(nbuf):
                j = j0 + t
                b = (1 + t) % nbuf
                bp = t % nbuf
                gather_wait(b)
                store(j, b)
                store_wait(bp)
                gather(j + nbuf - 1, bp)
            return carry

        lax.fori_loop(0, (n_chunks - nbuf) // nbuf, group, 0)

        for j in range(n_chunks - nbuf + 1, n_chunks):
            b = j % nbuf
            gather_wait(b)
            store(j, b)
        for b in range(nbuf):
            store_wait(b)

    return emb


def kernel(x, weight):
    b, h = x.shape
    n_vocab, dim = weight.shape
    n_flat = b * h
    info = plsc.get_sparse_core_info()
    num_workers = info.num_cores * info.num_subcores
    emb = _build(n_flat, dim, num_workers, chunk=320, nbuf=4)
    out = emb(weight, x.reshape(n_flat))
    return out.reshape(b, h, dim)
